# Initial kernel scaffold; baseline (speedup 1.0000x reference)
import functools

import jax
import jax.numpy as jnp
from jax import lax
from jax.experimental import pallas as pl
from jax.experimental.pallas import tpu as pltpu
from jax.experimental.pallas import tpu_sc as plsc

N = 10000
E = 160000
IN, HID, OUT = 256, 512, 256
ALPHA = 0.5
H = 128
OFF = N + 16
TROWS = 2 * N + 32
NT = 16
EPT = E // NT
CH = 79
CPT = CH * 128
PAD = CPT - EPT
ACC_R = N + 16

_mesh = plsc.VectorSubcoreMesh(core_axis_name="c", subcore_axis_name="s")


@functools.partial(
    pl.kernel,
    out_type=(
        jax.ShapeDtypeStruct((2 * N, H), jnp.float32),
        jax.ShapeDtypeStruct((N,), jnp.float32),
    ),
    mesh=_mesh,
    scratch_types=[
        pltpu.VMEM((CH, 128), jnp.int32),
        pltpu.VMEM((CH, 128), jnp.int32),
        pltpu.VMEM((128, H), jnp.float32),
        pltpu.VMEM((128,), jnp.float32),
        pltpu.VMEM((16, H), jnp.float32),
        pltpu.VMEM((640,), jnp.float32),
        pltpu.VMEM_SHARED((ACC_R, H), jnp.float32),
        pltpu.VMEM_SHARED((ACC_R,), jnp.float32),
        pltpu.SemaphoreType.DMA,
    ],
)
def _segsum(table, g_lo, g_hi, s_idx, out_sum, out_cnt,
            idxg, idxs, rows, ones, zrows, zflat, acc, cnt, sem):
    c = lax.axis_index("c")
    s = lax.axis_index("s")

    zv = jnp.zeros((16,), jnp.float32)
    ov = jnp.ones((16,), jnp.float32)
    for i in range(16):
        for k in range(H // 16):
            zrows[i, pl.ds(k * 16, 16)] = zv
    for k in range(128 // 16):
        ones[pl.ds(k * 16, 16)] = ov
    for k in range(640 // 16):
        zflat[pl.ds(k * 16, 16)] = zv

    pltpu.sync_copy(s_idx.at[s], idxs)

    @pl.when(c == 0)
    def _():
        pltpu.sync_copy(g_lo.at[s], idxg)

    @pl.when(c == 1)
    def _():
        pltpu.sync_copy(g_hi.at[s], idxg)

    base = s * 626

    def _zbody(j, carry):
        pltpu.sync_copy(zrows, acc.at[pl.ds(base + j * 16, 16)])
        return carry

    lax.fori_loop(0, 39, _zbody, 0)
    pltpu.sync_copy(zrows.at[pl.ds(0, 2)], acc.at[pl.ds(base + 624, 2)])

    @pl.when(s < 15)
    def _():
        pltpu.sync_copy(zflat, cnt.at[pl.ds(s * 640, 640)])

    @pl.when(s == 15)
    def _():
        pltpu.sync_copy(zflat.at[pl.ds(0, 416)], cnt.at[pl.ds(9600, 416)])

    plsc.subcore_barrier()

    def _body(ch, carry):
        pltpu.async_copy(table.at[idxg.at[ch]], rows, sem).wait()
        pltpu.sync_copy(rows, acc.at[idxs.at[ch]], add=True)
        pltpu.sync_copy(ones, cnt.at[idxs.at[ch]], add=True)
        return carry

    lax.fori_loop(0, CH, _body, 0)
    plsc.subcore_barrier()

    db = s * 625
    pltpu.sync_copy(acc.at[pl.ds(db, 625)], out_sum.at[pl.ds(c * N + db, 625)])

    @pl.when(c == 0)
    def _():
        @pl.when(s < 15)
        def _():
            pltpu.sync_copy(cnt.at[pl.ds(s * 640, 640)],
                            out_cnt.at[pl.ds(s * 640, 640)])

        @pl.when(s == 15)
        def _():
            pltpu.sync_copy(cnt.at[pl.ds(9600, 400)],
                            out_cnt.at[pl.ds(9600, 400)])


M1 = 1000


def _tc1_body(x, slo, shi, ulo, uhi, cin, cout, ws, wi, wo, b, h):
    rin = 1.0 / jnp.maximum(cin[...], 1.0)
    rout = 1.0 / jnp.maximum(cout[...], 1.0)
    tin = (jnp.dot(slo[...], wi[0:H, :], preferred_element_type=jnp.float32)
           + jnp.dot(shi[...], wi[H:IN, :], preferred_element_type=jnp.float32))
    tout = (jnp.dot(ulo[...], wo[0:H, :], preferred_element_type=jnp.float32)
            + jnp.dot(uhi[...], wo[H:IN, :], preferred_element_type=jnp.float32))
    hs = jnp.dot(x[...], ws[...], preferred_element_type=jnp.float32)
    h[...] = jnp.maximum(hs + tin * rin + tout * rout + b[...], 0.0)


def _tc2_body(h, wcat, s2, plo, phi, qlo, qhi):
    g = jnp.dot(h[...], wcat[...], preferred_element_type=jnp.float32)
    s2[...] = g[:, 0:OUT]
    plo[...] = g[:, OUT:OUT + H]
    phi[...] = g[:, OUT + H:OUT + 2 * H]
    qlo[...] = g[:, OUT + 2 * H:OUT + 3 * H]
    qhi[...] = g[:, OUT + 3 * H:OUT + 4 * H]


def _tc3_body(s2, slo, shi, ulo, uhi, cin, cout, b, out):
    rin = 1.0 / jnp.maximum(cin[...], 1.0)
    rout = 1.0 / jnp.maximum(cout[...], 1.0)
    lo = slo[...] * rin + ulo[...] * rout
    hi = shi[...] * rin + uhi[...] * rout
    out[...] = s2[...] + jnp.concatenate([lo, hi], axis=1) + b[...]


def _row_spec(w):
    return pl.BlockSpec((M1, w), lambda i: (i, 0))


def _full_spec(r, w):
    return pl.BlockSpec((r, w), lambda i: (0, 0))


def kernel(x, edge_index, W_in1, b_in1, W_out1, b_out1, W_self1, b_self1,
           W_in2, b_in2, W_out2, b_out2, W_self2, b_self2):
    f32 = jnp.float32
    src = edge_index[0]
    dst = edge_index[1]

    pad = (N + (jnp.arange(PAD, dtype=jnp.int32) % 16))[None, :]
    pad = jnp.broadcast_to(pad, (NT, PAD))

    def mk(v):
        t = jnp.concatenate([v.reshape(NT, EPT), pad], axis=1)
        return t.reshape(NT, CH, 128)

    src_lo = mk(src)
    dst_lo = mk(dst)
    src_hi = src_lo + OFF
    dst_hi = dst_lo + OFF

    z16 = jnp.zeros((16, H), f32)
    x2z = jnp.concatenate([x[:, :H], z16, x[:, H:], z16], axis=0)

    s1, cin = _segsum(x2z, src_lo, src_hi, dst_lo)
    u1, cout = _segsum(x2z, dst_lo, dst_hi, src_lo)

    cin2 = cin.reshape(N, 1)
    cout2 = cout.reshape(N, 1)

    wi1 = (1.0 - ALPHA) * W_in1
    wo1 = ALPHA * W_out1
    b1 = (b_self1 + (1.0 - ALPHA) * b_in1 + ALPHA * b_out1).reshape(1, HID)

    h = pl.pallas_call(
        _tc1_body,
        grid=(N // M1,),
        in_specs=[
            _row_spec(IN), _row_spec(H), _row_spec(H), _row_spec(H),
            _row_spec(H), _row_spec(1), _row_spec(1),
            _full_spec(IN, HID), _full_spec(IN, HID), _full_spec(IN, HID),
            _full_spec(1, HID),
        ],
        out_specs=_row_spec(HID),
        out_shape=jax.ShapeDtypeStruct((N, HID), f32),
    )(x, s1[0:N], s1[N:2 * N], u1[0:N], u1[N:2 * N], cin2, cout2,
      W_self1, wi1, wo1, b1)

    wcat = jnp.concatenate(
        [W_self2, (1.0 - ALPHA) * W_in2, ALPHA * W_out2], axis=1)
    s2, plo, phi, qlo, qhi = pl.pallas_call(
        _tc2_body,
        grid=(N // M1,),
        in_specs=[_row_spec(HID), _full_spec(HID, OUT + 4 * H)],
        out_specs=[_row_spec(OUT), _row_spec(H), _row_spec(H),
                   _row_spec(H), _row_spec(H)],
        out_shape=[
            jax.ShapeDtypeStruct((N, OUT), f32),
            jax.ShapeDtypeStruct((N, H), f32),
            jax.ShapeDtypeStruct((N, H), f32),
            jax.ShapeDtypeStruct((N, H), f32),
            jax.ShapeDtypeStruct((N, H), f32),
        ],
    )(h, wcat)

    p2z = jnp.concatenate([plo, z16, phi, z16], axis=0)
    q2z = jnp.concatenate([qlo, z16, qhi, z16], axis=0)

    s2sum, _ = _segsum(p2z, src_lo, src_hi, dst_lo)
    u2sum, _ = _segsum(q2z, dst_lo, dst_hi, src_lo)

    b2 = (b_self2 + (1.0 - ALPHA) * b_in2 + ALPHA * b_out2).reshape(1, OUT)
    out = pl.pallas_call(
        _tc3_body,
        grid=(N // M1,),
        in_specs=[
            _row_spec(OUT), _row_spec(H), _row_spec(H), _row_spec(H),
            _row_spec(H), _row_spec(1), _row_spec(1), _full_spec(1, OUT),
        ],
        out_specs=_row_spec(OUT),
        out_shape=jax.ShapeDtypeStruct((N, OUT), f32),
    )(s2, s2sum[0:N], s2sum[N:2 * N], u2sum[0:N], u2sum[N:2 * N],
      cin2, cout2, b2)
    return out


# speedup vs baseline: 5.5286x; 5.5286x over previous
import functools

import jax
import jax.numpy as jnp
from jax import lax
from jax.experimental import pallas as pl
from jax.experimental.pallas import tpu as pltpu
from jax.experimental.pallas import tpu_sc as plsc

N = 10000
E = 160000
IN, HID, OUT = 256, 512, 256
ALPHA = 0.5
H = 128
OFF = N + 16
TROWS = 2 * N + 32
NT = 16
EPT = E // NT
CH = 79
CPT = CH * 128
PAD = CPT - EPT
ACC_R = N + 16

_mesh = plsc.VectorSubcoreMesh(core_axis_name="c", subcore_axis_name="s")


@functools.partial(
    pl.kernel,
    out_type=(
        jax.ShapeDtypeStruct((2 * N, H), jnp.float32),
        jax.ShapeDtypeStruct((N,), jnp.float32),
    ),
    mesh=_mesh,
    scratch_types=[
        pltpu.VMEM((CH, 128), jnp.int32),
        pltpu.VMEM((CH, 128), jnp.int32),
        pltpu.VMEM((128, H), jnp.float32),
        pltpu.VMEM((128,), jnp.float32),
        pltpu.VMEM((16, H), jnp.float32),
        pltpu.VMEM((640,), jnp.float32),
        pltpu.VMEM_SHARED((ACC_R, H), jnp.float32),
        pltpu.VMEM_SHARED((ACC_R,), jnp.float32),
        pltpu.SemaphoreType.DMA,
    ],
)
def _segsum(table, g_lo, g_hi, s_idx, out_sum, out_cnt,
            idxg, idxs, rows, ones, zrows, zflat, acc, cnt, sem):
    c = lax.axis_index("c")
    s = lax.axis_index("s")

    zv = jnp.zeros((16,), jnp.float32)
    ov = jnp.ones((16,), jnp.float32)
    for i in range(16):
        for k in range(H // 16):
            zrows[i, pl.ds(k * 16, 16)] = zv
    for k in range(128 // 16):
        ones[pl.ds(k * 16, 16)] = ov
    for k in range(640 // 16):
        zflat[pl.ds(k * 16, 16)] = zv

    pltpu.sync_copy(s_idx.at[s], idxs)

    @pl.when(c == 0)
    def _():
        pltpu.sync_copy(g_lo.at[s], idxg)

    @pl.when(c == 1)
    def _():
        pltpu.sync_copy(g_hi.at[s], idxg)

    base = s * 632
    nfull = jnp.where(s < 15, 39, 33)

    def _zbody(j, carry):
        pltpu.sync_copy(zrows, acc.at[pl.ds(base + j * 16, 16)])
        return carry

    lax.fori_loop(0, nfull, _zbody, 0)
    pltpu.sync_copy(zrows.at[pl.ds(0, 8)], acc.at[pl.ds(base + nfull * 16, 8)])

    @pl.when(s < 15)
    def _():
        pltpu.sync_copy(zflat, cnt.at[pl.ds(s * 640, 640)])

    @pl.when(s == 15)
    def _():
        pltpu.sync_copy(zflat.at[pl.ds(0, 416)], cnt.at[pl.ds(9600, 416)])

    plsc.subcore_barrier()

    def _body(ch, carry):
        pltpu.async_copy(table.at[idxg.at[ch]], rows, sem).wait()
        pltpu.sync_copy(rows, acc.at[idxs.at[ch]], add=True)
        pltpu.sync_copy(ones, cnt.at[idxs.at[ch]], add=True)
        return carry

    lax.fori_loop(0, CH, _body, 0)
    plsc.subcore_barrier()

    db = s * 632

    @pl.when(s < 15)
    def _():
        pltpu.sync_copy(acc.at[pl.ds(db, 632)],
                        out_sum.at[pl.ds(c * N + db, 632)])

    @pl.when(s == 15)
    def _():
        pltpu.sync_copy(acc.at[pl.ds(9480, 520)],
                        out_sum.at[pl.ds(c * N + 9480, 520)])

    @pl.when(c == 0)
    def _():
        @pl.when(s < 15)
        def _():
            pltpu.sync_copy(cnt.at[pl.ds(s * 640, 640)], zflat)
            pltpu.sync_copy(zflat, out_cnt.at[pl.ds(s * 640, 640)])

        @pl.when(s == 15)
        def _():
            pltpu.sync_copy(cnt.at[pl.ds(9600, 400)], zflat.at[pl.ds(0, 400)])
            pltpu.sync_copy(zflat.at[pl.ds(0, 400)], out_cnt.at[pl.ds(9600, 400)])


M1 = 1000


def _tc1_body(x, slo, shi, ulo, uhi, cin, cout, ws, wi, wo, b, h):
    rin = 1.0 / jnp.maximum(cin[...], 1.0)
    rout = 1.0 / jnp.maximum(cout[...], 1.0)
    tin = (jnp.dot(slo[...], wi[0:H, :], preferred_element_type=jnp.float32)
           + jnp.dot(shi[...], wi[H:IN, :], preferred_element_type=jnp.float32))
    tout = (jnp.dot(ulo[...], wo[0:H, :], preferred_element_type=jnp.float32)
            + jnp.dot(uhi[...], wo[H:IN, :], preferred_element_type=jnp.float32))
    hs = jnp.dot(x[...], ws[...], preferred_element_type=jnp.float32)
    h[...] = jnp.maximum(hs + tin * rin + tout * rout + b[...], 0.0)


def _tc2_body(h, wcat, s2, plo, phi, qlo, qhi):
    g = jnp.dot(h[...], wcat[...], preferred_element_type=jnp.float32)
    s2[...] = g[:, 0:OUT]
    plo[...] = g[:, OUT:OUT + H]
    phi[...] = g[:, OUT + H:OUT + 2 * H]
    qlo[...] = g[:, OUT + 2 * H:OUT + 3 * H]
    qhi[...] = g[:, OUT + 3 * H:OUT + 4 * H]


def _tc3_body(s2, slo, shi, ulo, uhi, cin, cout, b, out):
    rin = 1.0 / jnp.maximum(cin[...], 1.0)
    rout = 1.0 / jnp.maximum(cout[...], 1.0)
    lo = slo[...] * rin + ulo[...] * rout
    hi = shi[...] * rin + uhi[...] * rout
    out[...] = s2[...] + jnp.concatenate([lo, hi], axis=1) + b[...]


def _row_spec(w):
    return pl.BlockSpec((M1, w), lambda i: (i, 0))


def _full_spec(r, w):
    return pl.BlockSpec((r, w), lambda i: (0, 0))


def kernel(x, edge_index, W_in1, b_in1, W_out1, b_out1, W_self1, b_self1,
           W_in2, b_in2, W_out2, b_out2, W_self2, b_self2):
    f32 = jnp.float32
    src = edge_index[0]
    dst = edge_index[1]

    pad = (N + (jnp.arange(PAD, dtype=jnp.int32) % 16))[None, :]
    pad = jnp.broadcast_to(pad, (NT, PAD))

    def mk(v):
        t = jnp.concatenate([v.reshape(NT, EPT), pad], axis=1)
        return t.reshape(NT, CH, 128)

    src_lo = mk(src)
    dst_lo = mk(dst)
    src_hi = src_lo + OFF
    dst_hi = dst_lo + OFF

    z16 = jnp.zeros((16, H), f32)
    x2z = jnp.concatenate([x[:, :H], z16, x[:, H:], z16], axis=0)

    s1, cin = _segsum(x2z, src_lo, src_hi, dst_lo)
    u1, cout = _segsum(x2z, dst_lo, dst_hi, src_lo)

    cin2 = cin.reshape(N, 1)
    cout2 = cout.reshape(N, 1)

    wi1 = (1.0 - ALPHA) * W_in1
    wo1 = ALPHA * W_out1
    b1 = (b_self1 + (1.0 - ALPHA) * b_in1 + ALPHA * b_out1).reshape(1, HID)

    h = pl.pallas_call(
        _tc1_body,
        grid=(N // M1,),
        in_specs=[
            _row_spec(IN), _row_spec(H), _row_spec(H), _row_spec(H),
            _row_spec(H), _row_spec(1), _row_spec(1),
            _full_spec(IN, HID), _full_spec(IN, HID), _full_spec(IN, HID),
            _full_spec(1, HID),
        ],
        out_specs=_row_spec(HID),
        out_shape=jax.ShapeDtypeStruct((N, HID), f32),
    )(x, s1[0:N], s1[N:2 * N], u1[0:N], u1[N:2 * N], cin2, cout2,
      W_self1, wi1, wo1, b1)

    wcat = jnp.concatenate(
        [W_self2, (1.0 - ALPHA) * W_in2, ALPHA * W_out2], axis=1)
    s2, plo, phi, qlo, qhi = pl.pallas_call(
        _tc2_body,
        grid=(N // M1,),
        in_specs=[_row_spec(HID), _full_spec(HID, OUT + 4 * H)],
        out_specs=[_row_spec(OUT), _row_spec(H), _row_spec(H),
                   _row_spec(H), _row_spec(H)],
        out_shape=[
            jax.ShapeDtypeStruct((N, OUT), f32),
            jax.ShapeDtypeStruct((N, H), f32),
            jax.ShapeDtypeStruct((N, H), f32),
            jax.ShapeDtypeStruct((N, H), f32),
            jax.ShapeDtypeStruct((N, H), f32),
        ],
    )(h, wcat)

    p2z = jnp.concatenate([plo, z16, phi, z16], axis=0)
    q2z = jnp.concatenate([qlo, z16, qhi, z16], axis=0)

    s2sum, _ = _segsum(p2z, src_lo, src_hi, dst_lo)
    u2sum, _ = _segsum(q2z, dst_lo, dst_hi, src_lo)

    b2 = (b_self2 + (1.0 - ALPHA) * b_in2 + ALPHA * b_out2).reshape(1, OUT)
    out = pl.pallas_call(
        _tc3_body,
        grid=(N // M1,),
        in_specs=[
            _row_spec(OUT), _row_spec(H), _row_spec(H), _row_spec(H),
            _row_spec(H), _row_spec(1), _row_spec(1), _full_spec(1, OUT),
        ],
        out_specs=_row_spec(OUT),
        out_shape=jax.ShapeDtypeStruct((N, OUT), f32),
    )(s2, s2sum[0:N], s2sum[N:2 * N], u2sum[0:N], u2sum[N:2 * N],
      cin2, cout2, b2)
    return out


# speedup vs baseline: 7.8903x; 1.4272x over previous
import functools

import jax
import jax.numpy as jnp
from jax import lax
from jax.experimental import pallas as pl
from jax.experimental.pallas import tpu as pltpu
from jax.experimental.pallas import tpu_sc as plsc

N = 10000
E = 160000
IN, HID, OUT = 256, 512, 256
ALPHA = 0.5
H = 128
OFF = N + 16
TROWS = 2 * N + 32
NT = 16
EPT = E // NT
CH = 80
CPT = CH * 128
PAD = CPT - EPT
ACC_R = N + 16

_mesh = plsc.VectorSubcoreMesh(core_axis_name="c", subcore_axis_name="s")


def _make_segsum(with_counts):
    if with_counts:
        out_type = (
            jax.ShapeDtypeStruct((2 * N, H), jnp.float32),
            jax.ShapeDtypeStruct((N,), jnp.float32),
        )
    else:
        out_type = jax.ShapeDtypeStruct((2 * N, H), jnp.float32)

    @functools.partial(
        pl.kernel,
        out_type=out_type,
        mesh=_mesh,
        scratch_types=[
            pltpu.VMEM((CH // 2, 128), jnp.int32),
            pltpu.VMEM((CH // 2, 128), jnp.int32),
            pltpu.VMEM((128, H), jnp.float32),
            pltpu.VMEM((128, H), jnp.float32),
            pltpu.VMEM((128,), jnp.float32),
            pltpu.VMEM((640,), jnp.float32),
            pltpu.VMEM_SHARED((ACC_R, H), jnp.float32),
            pltpu.VMEM_SHARED((ACC_R,), jnp.float32),
            pltpu.SemaphoreType.DMA,
            pltpu.SemaphoreType.DMA,
        ],
    )
    def _segsum(table, g_lo, g_hi, s_idx, *rest):
        if with_counts:
            (out_sum, out_cnt, idxg, idxs, rows_a, rows_b, ones,
             zflat, acc, cnt, sem_a, sem_b) = rest
        else:
            (out_sum, idxg, idxs, rows_a, rows_b, ones,
             zflat, acc, cnt, sem_a, sem_b) = rest
        c = lax.axis_index("c")
        s = lax.axis_index("s")
        HC = CH // 2

        zv = jnp.zeros((16,), jnp.float32)
        ov = jnp.ones((16,), jnp.float32)

        def _zr(i, carry):
            for k in range(H // 16):
                rows_a[i, pl.ds(k * 16, 16)] = zv
            return carry

        lax.fori_loop(0, 128, _zr, 0)
        if with_counts:
            for k in range(128 // 16):
                ones[pl.ds(k * 16, 16)] = ov
            for k in range(640 // 16):
                zflat[pl.ds(k * 16, 16)] = zv

        base = s * 632
        for j in range(4):
            pltpu.sync_copy(rows_a, acc.at[pl.ds(base + j * 128, 128)])

        @pl.when(s < 15)
        def _():
            pltpu.sync_copy(rows_a.at[pl.ds(0, 120)],
                            acc.at[pl.ds(base + 512, 120)])

        @pl.when(s == 15)
        def _():
            pltpu.sync_copy(rows_a.at[pl.ds(0, 24)],
                            acc.at[pl.ds(base + 512, 24)])

        if with_counts:
            @pl.when(s < 15)
            def _():
                pltpu.sync_copy(zflat, cnt.at[pl.ds(s * 640, 640)])

            @pl.when(s == 15)
            def _():
                pltpu.sync_copy(zflat.at[pl.ds(0, 416)],
                                cnt.at[pl.ds(9600, 416)])

        plsc.subcore_barrier()

        def _gather(ch, buf, sem):
            return pltpu.async_copy(table.at[idxg.at[ch]], buf, sem)

        def _drain(ch, buf, sem):
            pltpu.make_async_copy(table.at[idxg.at[ch]], buf, sem).wait()

        def _scatter(ch, buf):
            pltpu.sync_copy(buf, acc.at[idxs.at[ch]], add=True)
            if with_counts:
                pltpu.sync_copy(ones, cnt.at[idxs.at[ch]], add=True)

        for half in range(2):
            pltpu.sync_copy(s_idx.at[s, pl.ds(half * HC, HC)], idxs)

            @pl.when(c == 0)
            def _():
                pltpu.sync_copy(g_lo.at[s, pl.ds(half * HC, HC)], idxg)

            @pl.when(c == 1)
            def _():
                pltpu.sync_copy(g_hi.at[s, pl.ds(half * HC, HC)], idxg)

            _gather(0, rows_a, sem_a)

            def _body(j, carry):
                a = 2 * j
                b = a + 1
                _gather(b, rows_b, sem_b)
                _drain(a, rows_a, sem_a)
                _scatter(a, rows_a)

                @pl.when(j < HC // 2 - 1)
                def _():
                    _gather(a + 2, rows_a, sem_a)

                _drain(b, rows_b, sem_b)
                _scatter(b, rows_b)
                return carry

            lax.fori_loop(0, HC // 2, _body, 0)

        plsc.subcore_barrier()

        db = s * 632

        @pl.when(s < 15)
        def _():
            pltpu.sync_copy(acc.at[pl.ds(db, 632)],
                            out_sum.at[pl.ds(c * N + db, 632)])

        @pl.when(s == 15)
        def _():
            pltpu.sync_copy(acc.at[pl.ds(9480, 520)],
                            out_sum.at[pl.ds(c * N + 9480, 520)])

        if with_counts:
            @pl.when(c == 0)
            def _():
                @pl.when(s < 15)
                def _():
                    pltpu.sync_copy(cnt.at[pl.ds(s * 640, 640)], zflat)
                    pltpu.sync_copy(zflat, out_cnt.at[pl.ds(s * 640, 640)])

                @pl.when(s == 15)
                def _():
                    pltpu.sync_copy(cnt.at[pl.ds(9600, 400)],
                                    zflat.at[pl.ds(0, 400)])
                    pltpu.sync_copy(zflat.at[pl.ds(0, 400)],
                                    out_cnt.at[pl.ds(9600, 400)])

    return _segsum


_segsum_cnt = _make_segsum(True)


M1 = 1000


def _tc1_body(x, slo, shi, ulo, uhi, cin, cout, ws, wi, wo, b, h):
    rin = 1.0 / jnp.maximum(cin[...], 1.0)
    rout = 1.0 / jnp.maximum(cout[...], 1.0)
    tin = (jnp.dot(slo[...], wi[0:H, :], preferred_element_type=jnp.float32)
           + jnp.dot(shi[...], wi[H:IN, :], preferred_element_type=jnp.float32))
    tout = (jnp.dot(ulo[...], wo[0:H, :], preferred_element_type=jnp.float32)
            + jnp.dot(uhi[...], wo[H:IN, :], preferred_element_type=jnp.float32))
    hs = jnp.dot(x[...], ws[...], preferred_element_type=jnp.float32)
    h[...] = jnp.maximum(hs + tin * rin + tout * rout + b[...], 0.0)


def _tc2_body(h, wcat, s2, plo, phi, qlo, qhi):
    g = jnp.dot(h[...], wcat[...], preferred_element_type=jnp.float32)
    s2[...] = g[:, 0:OUT]
    plo[...] = g[:, OUT:OUT + H]
    phi[...] = g[:, OUT + H:OUT + 2 * H]
    qlo[...] = g[:, OUT + 2 * H:OUT + 3 * H]
    qhi[...] = g[:, OUT + 3 * H:OUT + 4 * H]


def _tc3_body(s2, slo, shi, ulo, uhi, cin, cout, b, out):
    rin = 1.0 / jnp.maximum(cin[...], 1.0)
    rout = 1.0 / jnp.maximum(cout[...], 1.0)
    lo = slo[...] * rin + ulo[...] * rout
    hi = shi[...] * rin + uhi[...] * rout
    out[...] = s2[...] + jnp.concatenate([lo, hi], axis=1) + b[...]


def _row_spec(w):
    return pl.BlockSpec((M1, w), lambda i: (i, 0))


def _full_spec(r, w):
    return pl.BlockSpec((r, w), lambda i: (0, 0))


def kernel(x, edge_index, W_in1, b_in1, W_out1, b_out1, W_self1, b_self1,
           W_in2, b_in2, W_out2, b_out2, W_self2, b_self2):
    f32 = jnp.float32
    src = edge_index[0]
    dst = edge_index[1]

    pad = (N + (jnp.arange(PAD, dtype=jnp.int32) % 16))[None, :]
    pad = jnp.broadcast_to(pad, (NT, PAD))

    def mk(v):
        t = jnp.concatenate([v.reshape(NT, EPT), pad], axis=1)
        return t.reshape(NT, CH, 128)

    src_lo = mk(src)
    dst_lo = mk(dst)
    src_hi = src_lo + OFF
    dst_hi = dst_lo + OFF

    z16 = jnp.zeros((16, H), f32)
    x2z = jnp.concatenate([x[:, :H], z16, x[:, H:], z16], axis=0)

    s1, cin = _segsum_cnt(x2z, src_lo, src_hi, dst_lo)
    u1, cout = _segsum_cnt(x2z, dst_lo, dst_hi, src_lo)

    cin2 = cin.reshape(N, 1)
    cout2 = cout.reshape(N, 1)

    wi1 = (1.0 - ALPHA) * W_in1
    wo1 = ALPHA * W_out1
    b1 = (b_self1 + (1.0 - ALPHA) * b_in1 + ALPHA * b_out1).reshape(1, HID)

    h = pl.pallas_call(
        _tc1_body,
        grid=(N // M1,),
        in_specs=[
            _row_spec(IN), _row_spec(H), _row_spec(H), _row_spec(H),
            _row_spec(H), _row_spec(1), _row_spec(1),
            _full_spec(IN, HID), _full_spec(IN, HID), _full_spec(IN, HID),
            _full_spec(1, HID),
        ],
        out_specs=_row_spec(HID),
        out_shape=jax.ShapeDtypeStruct((N, HID), f32),
    )(x, s1[0:N], s1[N:2 * N], u1[0:N], u1[N:2 * N], cin2, cout2,
      W_self1, wi1, wo1, b1)

    wcat = jnp.concatenate(
        [W_self2, (1.0 - ALPHA) * W_in2, ALPHA * W_out2], axis=1)
    s2, plo, phi, qlo, qhi = pl.pallas_call(
        _tc2_body,
        grid=(N // M1,),
        in_specs=[_row_spec(HID), _full_spec(HID, OUT + 4 * H)],
        out_specs=[_row_spec(OUT), _row_spec(H), _row_spec(H),
                   _row_spec(H), _row_spec(H)],
        out_shape=[
            jax.ShapeDtypeStruct((N, OUT), f32),
            jax.ShapeDtypeStruct((N, H), f32),
            jax.ShapeDtypeStruct((N, H), f32),
            jax.ShapeDtypeStruct((N, H), f32),
            jax.ShapeDtypeStruct((N, H), f32),
        ],
    )(h, wcat)

    p2z = jnp.concatenate([plo, z16, phi, z16], axis=0)
    q2z = jnp.concatenate([qlo, z16, qhi, z16], axis=0)

    s2sum, _ = _segsum_cnt(p2z, src_lo, src_hi, dst_lo)
    u2sum, _ = _segsum_cnt(q2z, dst_lo, dst_hi, src_lo)

    b2 = (b_self2 + (1.0 - ALPHA) * b_in2 + ALPHA * b_out2).reshape(1, OUT)
    out = pl.pallas_call(
        _tc3_body,
        grid=(N // M1,),
        in_specs=[
            _row_spec(OUT), _row_spec(H), _row_spec(H), _row_spec(H),
            _row_spec(H), _row_spec(1), _row_spec(1), _full_spec(1, OUT),
        ],
        out_specs=_row_spec(OUT),
        out_shape=jax.ShapeDtypeStruct((N, OUT), f32),
    )(s2, s2sum[0:N], s2sum[N:2 * N], u2sum[0:N], u2sum[N:2 * N],
      cin2, cout2, b2)
    return out


# speedup vs baseline: 8.1339x; 1.0309x over previous
import functools

import jax
import jax.numpy as jnp
from jax import lax
from jax.experimental import pallas as pl
from jax.experimental.pallas import tpu as pltpu
from jax.experimental.pallas import tpu_sc as plsc

N = 10000
E = 160000
IN, HID, OUT = 256, 512, 256
ALPHA = 0.5
H = 128
NT = 16
EPT = E // NT
CH = 80
CPT = CH * 128
PAD = CPT - EPT
ACC_R = N + 16

_mesh = plsc.VectorSubcoreMesh(core_axis_name="c", subcore_axis_name="s")


def _make_segsum(with_counts):
    if with_counts:
        out_type = (
            jax.ShapeDtypeStruct((2 * N, H), jnp.float32),
            jax.ShapeDtypeStruct((N,), jnp.float32),
        )
    else:
        out_type = jax.ShapeDtypeStruct((2 * N, H), jnp.float32)

    @functools.partial(
        pl.kernel,
        out_type=out_type,
        mesh=_mesh,
        scratch_types=[
            pltpu.VMEM((CH // 2, 128), jnp.int32),
            pltpu.VMEM((CH // 2, 128), jnp.int32),
            pltpu.VMEM((128, H), jnp.float32),
            pltpu.VMEM((128, H), jnp.float32),
            pltpu.VMEM((128,), jnp.float32),
            pltpu.VMEM((640,), jnp.float32),
            pltpu.VMEM_SHARED((ACC_R, H), jnp.float32),
            pltpu.VMEM_SHARED((ACC_R,), jnp.float32),
            pltpu.SemaphoreType.DMA,
            pltpu.SemaphoreType.DMA,
        ],
    )
    def _segsum(table, g_lo, g_hi, s_idx, *rest):
        if with_counts:
            (out_sum, out_cnt, idxg, idxs, rows_a, rows_b, ones,
             zflat, acc, cnt, sem_a, sem_b) = rest
        else:
            (out_sum, idxg, idxs, rows_a, rows_b, ones,
             zflat, acc, cnt, sem_a, sem_b) = rest
        c = lax.axis_index("c")
        s = lax.axis_index("s")
        HC = CH // 2

        zv = jnp.zeros((16,), jnp.float32)
        ov = jnp.ones((16,), jnp.float32)

        def _zr(i, carry):
            for k in range(H // 16):
                rows_a[i, pl.ds(k * 16, 16)] = zv
            return carry

        lax.fori_loop(0, 128, _zr, 0)
        if with_counts:
            for k in range(128 // 16):
                ones[pl.ds(k * 16, 16)] = ov
            for k in range(640 // 16):
                zflat[pl.ds(k * 16, 16)] = zv

        base = s * 632
        for j in range(4):
            pltpu.sync_copy(rows_a, acc.at[pl.ds(base + j * 128, 128)])

        @pl.when(s < 15)
        def _():
            pltpu.sync_copy(rows_a.at[pl.ds(0, 120)],
                            acc.at[pl.ds(base + 512, 120)])

        @pl.when(s == 15)
        def _():
            pltpu.sync_copy(rows_a.at[pl.ds(0, 24)],
                            acc.at[pl.ds(base + 512, 24)])

        if with_counts:
            @pl.when(s < 15)
            def _():
                pltpu.sync_copy(zflat, cnt.at[pl.ds(s * 640, 640)])

            @pl.when(s == 15)
            def _():
                pltpu.sync_copy(zflat.at[pl.ds(0, 416)],
                                cnt.at[pl.ds(9600, 416)])

        plsc.subcore_barrier()

        def _gather(ch, buf, sem):
            return pltpu.async_copy(table.at[idxg.at[ch]], buf, sem)

        def _drain(ch, buf, sem):
            pltpu.make_async_copy(table.at[idxg.at[ch]], buf, sem).wait()

        def _scatter(ch, buf):
            pltpu.sync_copy(buf, acc.at[idxs.at[ch]], add=True)
            if with_counts:
                pltpu.sync_copy(ones, cnt.at[idxs.at[ch]], add=True)

        for half in range(2):
            pltpu.sync_copy(s_idx.at[s, pl.ds(half * HC, HC)], idxs)

            @pl.when(c == 0)
            def _():
                pltpu.sync_copy(g_lo.at[s, pl.ds(half * HC, HC)], idxg)

            @pl.when(c == 1)
            def _():
                pltpu.sync_copy(g_hi.at[s, pl.ds(half * HC, HC)], idxg)

            _gather(0, rows_a, sem_a)

            def _body(j, carry):
                a = 2 * j
                b = a + 1
                _gather(b, rows_b, sem_b)
                _drain(a, rows_a, sem_a)
                _scatter(a, rows_a)

                @pl.when(j < HC // 2 - 1)
                def _():
                    _gather(a + 2, rows_a, sem_a)

                _drain(b, rows_b, sem_b)
                _scatter(b, rows_b)
                return carry

            lax.fori_loop(0, HC // 2, _body, 0)

        plsc.subcore_barrier()

        db = s * 632

        @pl.when(s < 15)
        def _():
            pltpu.sync_copy(acc.at[pl.ds(db, 632)],
                            out_sum.at[pl.ds(c * N + db, 632)])

        @pl.when(s == 15)
        def _():
            pltpu.sync_copy(acc.at[pl.ds(9480, 520)],
                            out_sum.at[pl.ds(c * N + 9480, 520)])

        if with_counts:
            @pl.when(c == 0)
            def _():
                @pl.when(s < 15)
                def _():
                    pltpu.sync_copy(cnt.at[pl.ds(s * 640, 640)], zflat)
                    pltpu.sync_copy(zflat, out_cnt.at[pl.ds(s * 640, 640)])

                @pl.when(s == 15)
                def _():
                    pltpu.sync_copy(cnt.at[pl.ds(9600, 400)],
                                    zflat.at[pl.ds(0, 400)])
                    pltpu.sync_copy(zflat.at[pl.ds(0, 400)],
                                    out_cnt.at[pl.ds(9600, 400)])

    return _segsum


_segsum_cnt = _make_segsum(True)
_segsum_nocnt = _make_segsum(False)


M1 = 1000


def _tc1_body(x, slo, shi, ulo, uhi, cin, cout, ws, wi, wo, b, h):
    rin = 1.0 / jnp.maximum(cin[...], 1.0)
    rout = 1.0 / jnp.maximum(cout[...], 1.0)
    tin = (jnp.dot(slo[...], wi[0:H, :], preferred_element_type=jnp.float32)
           + jnp.dot(shi[...], wi[H:IN, :], preferred_element_type=jnp.float32))
    tout = (jnp.dot(ulo[...], wo[0:H, :], preferred_element_type=jnp.float32)
            + jnp.dot(uhi[...], wo[H:IN, :], preferred_element_type=jnp.float32))
    hs = jnp.dot(x[...], ws[...], preferred_element_type=jnp.float32)
    h[...] = jnp.maximum(hs + tin * rin + tout * rout + b[...], 0.0)


def _tc2_body(h, wcat, s2, p, q):
    g = jnp.dot(h[...], wcat[...], preferred_element_type=jnp.float32)
    s2[...] = g[:, 0:OUT]
    p[...] = g[:, OUT:2 * OUT]
    q[...] = g[:, 2 * OUT:3 * OUT]


def _tc3_body(s2, slo, shi, ulo, uhi, cin, cout, b, out):
    rin = 1.0 / jnp.maximum(cin[...], 1.0)
    rout = 1.0 / jnp.maximum(cout[...], 1.0)
    lo = slo[...] * rin + ulo[...] * rout
    hi = shi[...] * rin + uhi[...] * rout
    out[...] = s2[...] + jnp.concatenate([lo, hi], axis=1) + b[...]


def _row_spec(w):
    return pl.BlockSpec((M1, w), lambda i: (i, 0))


def _full_spec(r, w):
    return pl.BlockSpec((r, w), lambda i: (0, 0))


def kernel(x, edge_index, W_in1, b_in1, W_out1, b_out1, W_self1, b_self1,
           W_in2, b_in2, W_out2, b_out2, W_self2, b_self2):
    f32 = jnp.float32
    src = edge_index[0]
    dst = edge_index[1]

    spread = jnp.arange(PAD, dtype=jnp.int32) % 16

    def mk(v, padv):
        t = jnp.concatenate(
            [v.reshape(NT, EPT),
             jnp.broadcast_to(padv[None, :], (NT, PAD))], axis=1)
        return t.reshape(NT, CH, 128)

    src_g = mk(2 * src, 2 * spread)
    dst_g = mk(2 * dst, 2 * spread)
    src_s = mk(src, N + spread)
    dst_s = mk(dst, N + spread)
    src_g1 = src_g + 1
    dst_g1 = dst_g + 1

    x2 = x.reshape(2 * N, H)

    s1, cin = _segsum_cnt(x2, src_g, src_g1, dst_s)
    u1, cout = _segsum_cnt(x2, dst_g, dst_g1, src_s)

    cin2 = cin.reshape(N, 1)
    cout2 = cout.reshape(N, 1)

    wi1 = (1.0 - ALPHA) * W_in1
    wo1 = ALPHA * W_out1
    b1 = (b_self1 + (1.0 - ALPHA) * b_in1 + ALPHA * b_out1).reshape(1, HID)

    h = pl.pallas_call(
        _tc1_body,
        grid=(N // M1,),
        in_specs=[
            _row_spec(IN), _row_spec(H), _row_spec(H), _row_spec(H),
            _row_spec(H), _row_spec(1), _row_spec(1),
            _full_spec(IN, HID), _full_spec(IN, HID), _full_spec(IN, HID),
            _full_spec(1, HID),
        ],
        out_specs=_row_spec(HID),
        out_shape=jax.ShapeDtypeStruct((N, HID), f32),
    )(x, s1[0:N], s1[N:2 * N], u1[0:N], u1[N:2 * N], cin2, cout2,
      W_self1, wi1, wo1, b1)

    wcat = jnp.concatenate(
        [W_self2, (1.0 - ALPHA) * W_in2, ALPHA * W_out2], axis=1)
    s2, p, q = pl.pallas_call(
        _tc2_body,
        grid=(N // M1,),
        in_specs=[_row_spec(HID), _full_spec(HID, 3 * OUT)],
        out_specs=[_row_spec(OUT), _row_spec(OUT), _row_spec(OUT)],
        out_shape=[
            jax.ShapeDtypeStruct((N, OUT), f32),
            jax.ShapeDtypeStruct((N, OUT), f32),
            jax.ShapeDtypeStruct((N, OUT), f32),
        ],
    )(h, wcat)

    s2sum = _segsum_nocnt(p.reshape(2 * N, H), src_g, src_g1, dst_s)
    u2sum = _segsum_nocnt(q.reshape(2 * N, H), dst_g, dst_g1, src_s)

    b2 = (b_self2 + (1.0 - ALPHA) * b_in2 + ALPHA * b_out2).reshape(1, OUT)
    out = pl.pallas_call(
        _tc3_body,
        grid=(N // M1,),
        in_specs=[
            _row_spec(OUT), _row_spec(H), _row_spec(H), _row_spec(H),
            _row_spec(H), _row_spec(1), _row_spec(1), _full_spec(1, OUT),
        ],
        out_specs=_row_spec(OUT),
        out_shape=jax.ShapeDtypeStruct((N, OUT), f32),
    )(s2, s2sum[0:N], s2sum[N:2 * N], u2sum[0:N], u2sum[N:2 * N],
      cin2, cout2, b2)
    return out
